# baseline (speedup 1.0000x reference)
import functools

import jax
import jax.numpy as jnp
from jax import lax
from jax.experimental import pallas as pl
from jax.experimental.pallas import tpu as pltpu
from jax.experimental.pallas import tpu_sc as plsc

_NC = 2
_NS = 16
_NW = _NC * _NS

_N_ROWS = 100000
_D = 128
_N_CLASSES = 256
_CHUNK = 160
_N_CHUNKS = _N_ROWS // _CHUNK
_CHUNKS_PER_TILE = -(-_N_CHUNKS // _NW)
_UNROLL = _NS


def _sc_segment_sums(z_support, y_support):
    mesh = plsc.VectorSubcoreMesh(core_axis_name="c", subcore_axis_name="s")

    @functools.partial(
        pl.kernel,
        out_type=(
            jax.ShapeDtypeStruct((_NW, _N_CLASSES, _D), jnp.float32),
            jax.ShapeDtypeStruct((_NW, _N_CLASSES, _NS), jnp.float32),
        ),
        mesh=mesh,
        scratch_types=dict(
            rows0=pltpu.VMEM((_CHUNK, _D), jnp.float32),
            rows1=pltpu.VMEM((_CHUNK, _D), jnp.float32),
            lab0=pltpu.VMEM((_CHUNK,), jnp.int32),
            lab1=pltpu.VMEM((_CHUNK,), jnp.int32),
            acc=pltpu.VMEM((_N_CLASSES, _D), jnp.float32),
            cacc=pltpu.VMEM((_N_CLASSES, _NS), jnp.float32),
            sem_r0=pltpu.SemaphoreType.DMA,
            sem_r1=pltpu.SemaphoreType.DMA,
            sem_l0=pltpu.SemaphoreType.DMA,
            sem_l1=pltpu.SemaphoreType.DMA,
        ),
    )
    def k(z_hbm, y_hbm, sums_hbm, cnts_hbm, *, rows0, rows1, lab0, lab1,
          acc, cacc, sem_r0, sem_r1, sem_l0, sem_l1):
        c_idx = lax.axis_index("c")
        s_idx = lax.axis_index("s")
        wid = s_idx * _NC + c_idx

        zero16 = jnp.zeros((_NS,), jnp.float32)
        one16 = jnp.ones((_NS,), jnp.float32)

        def zero_acc(r, carry):
            for kk in range(_D // _NS):
                acc[r, pl.ds(kk * _NS, _NS)] = zero16
            cacc[r, :] = zero16
            return carry

        lax.fori_loop(0, _N_CLASSES, zero_acc, 0)

        bufs = ((rows0, lab0, sem_r0, sem_l0), (rows1, lab1, sem_r1, sem_l1))

        def issue(b, c):
            rows, lab, sem_r, sem_l = bufs[b]

            @pl.when(c < _N_CHUNKS)
            def _():
                pltpu.async_copy(z_hbm.at[pl.ds(c * _CHUNK, _CHUNK)], rows, sem_r)
                pltpu.async_copy(y_hbm.at[pl.ds(c * _CHUNK, _CHUNK)], lab, sem_l)

        def consume(b, c):
            rows, lab, sem_r, sem_l = bufs[b]

            @pl.when(c < _N_CHUNKS)
            def _():
                pltpu.make_async_copy(z_hbm.at[pl.ds(c * _CHUNK, _CHUNK)], rows, sem_r).wait()
                pltpu.make_async_copy(y_hbm.at[pl.ds(c * _CHUNK, _CHUNK)], lab, sem_l).wait()

                def rows_body(rr, carry):
                    labv = lab[pl.ds(rr * _UNROLL, _NS)]
                    for u in range(_UNROLL):
                        r = rr * _UNROLL + u
                        lbl = labv[u]
                        for kk in range(_D // _NS):
                            plsc.addupdate(
                                acc.at[lbl, pl.ds(kk * _NS, _NS)],
                                rows[r, pl.ds(kk * _NS, _NS)])
                        plsc.addupdate(cacc.at[lbl], one16)
                    return carry

                lax.fori_loop(0, _CHUNK // _UNROLL, rows_body, 0)

        issue(0, wid)
        issue(1, wid + _NW)

        def outer(i2, carry):
            for b in range(2):
                c = wid + (2 * i2 + b) * _NW
                consume(b, c)
                issue(b, c + 2 * _NW)
            return carry

        lax.fori_loop(0, _CHUNKS_PER_TILE // 2, outer, 0)

        pltpu.sync_copy(acc, sums_hbm.at[wid])
        pltpu.sync_copy(cacc, cnts_hbm.at[wid])

    return k(z_support, y_support)


def _tc_body(psums_ref, pcnts_ref, q_ref, tau_ref, out_ref):
    sums = psums_ref[0]
    cnts = pcnts_ref[0]
    for w in range(1, _NW):
        sums = sums + psums_ref[w]
        cnts = cnts + pcnts_ref[w]
    cnt = cnts[:, 0:1]
    protos = sums / cnt
    q = q_ref[...]
    qn = jnp.sum(q * q, axis=1, keepdims=True)
    pn = jnp.sum(protos * protos, axis=1)[None, :]
    cross = lax.dot_general(q, protos, (((1,), (1,)), ((), ())))
    out_ref[...] = (-tau_ref[0, 0]) * (qn - 2.0 * cross + pn)


def kernel(z_support, y_support, z_query, tau):
    psums, pcnts = _sc_segment_sums(z_support, y_support)
    tau2d = jnp.asarray(tau, jnp.float32).reshape(1, 1)
    logits = pl.pallas_call(
        _tc_body,
        out_shape=jax.ShapeDtypeStruct((z_query.shape[0], _N_CLASSES), jnp.float32),
        in_specs=[
            pl.BlockSpec(memory_space=pltpu.VMEM),
            pl.BlockSpec(memory_space=pltpu.VMEM),
            pl.BlockSpec(memory_space=pltpu.VMEM),
            pl.BlockSpec(memory_space=pltpu.SMEM),
        ],
        out_specs=pl.BlockSpec(memory_space=pltpu.VMEM),
    )(psums, pcnts, z_query, tau2d)
    return logits


# speedup vs baseline: 2.1706x; 2.1706x over previous
import functools

import jax
import jax.numpy as jnp
from jax import lax
from jax.experimental import pallas as pl
from jax.experimental.pallas import tpu as pltpu
from jax.experimental.pallas import tpu_sc as plsc

_NC = 2
_NS = 16
_NW = _NC * _NS

_N_ROWS = 100000
_D = 128
_N_CLASSES = 256
_CHUNK = 400
_GROUP = 80
_N_GROUPS = _CHUNK // _GROUP
_N_CHUNKS = _N_ROWS // _CHUNK
_CHUNKS_PER_TILE = -(-_N_CHUNKS // _NW)


def _sc_segment_sums(z_support, y3d):
    mesh = plsc.VectorSubcoreMesh(core_axis_name="c", subcore_axis_name="s")

    @functools.partial(
        pl.kernel,
        out_type=(
            jax.ShapeDtypeStruct((2 * _NC, _N_CLASSES, _D), jnp.float32),
            jax.ShapeDtypeStruct((_NC, _N_CLASSES, _NS), jnp.float32),
        ),
        mesh=mesh,
        scratch_types=dict(
            rows0=pltpu.VMEM((_CHUNK, _D), jnp.float32),
            rows1=pltpu.VMEM((_CHUNK, _D), jnp.float32),
            lab0=pltpu.VMEM((_CHUNK,), jnp.int32),
            lab1=pltpu.VMEM((_CHUNK,), jnp.int32),
            labq0=pltpu.VMEM((_N_GROUPS, _GROUP), jnp.int32),
            labq1=pltpu.VMEM((_N_GROUPS, _GROUP), jnp.int32),
            ones_buf=pltpu.VMEM((_GROUP, _NS), jnp.float32),
            zrow=pltpu.VMEM((_NS, _D), jnp.float32),
            zrow16=pltpu.VMEM((_NS, _NS), jnp.float32),
            acc=pltpu.VMEM_SHARED((_N_CLASSES, _D), jnp.float32),
            acc2=pltpu.VMEM_SHARED((_N_CLASSES, _D), jnp.float32),
            cacc=pltpu.VMEM_SHARED((_N_CLASSES, _NS), jnp.float32),
            sem_r0=pltpu.SemaphoreType.DMA,
            sem_r1=pltpu.SemaphoreType.DMA,
            sem_l0=pltpu.SemaphoreType.DMA,
            sem_l1=pltpu.SemaphoreType.DMA,
            sem_sc=pltpu.SemaphoreType.DMA,
        ),
    )
    def k(z_hbm, y_hbm, sums_hbm, cnts_hbm, *, rows0, rows1, lab0, lab1,
          labq0, labq1, ones_buf, zrow, zrow16, acc, acc2, cacc, sem_r0,
          sem_r1, sem_l0, sem_l1, sem_sc):
        c_idx = lax.axis_index("c")
        s_idx = lax.axis_index("s")
        wid = s_idx * _NC + c_idx

        zero16 = jnp.zeros((_NS,), jnp.float32)
        one16 = jnp.ones((_NS,), jnp.float32)

        def fill_zrow(r, carry):
            for kk in range(_D // _NS):
                zrow[r, pl.ds(kk * _NS, _NS)] = zero16
            zrow16[r, :] = zero16
            return carry

        lax.fori_loop(0, _NS, fill_zrow, 0)

        def fill_ones(r, carry):
            ones_buf[r, :] = one16
            return carry

        lax.fori_loop(0, _GROUP, fill_ones, 0)

        pltpu.sync_copy(zrow, acc.at[pl.ds(s_idx * _NS, _NS)])
        pltpu.sync_copy(zrow, acc2.at[pl.ds(s_idx * _NS, _NS)])
        pltpu.sync_copy(zrow16, cacc.at[pl.ds(s_idx * _NS, _NS)])
        plsc.subcore_barrier()

        bufs = ((rows0, lab0, labq0, sem_r0, sem_l0),
                (rows1, lab1, labq1, sem_r1, sem_l1))

        def chunk_id(i):
            return wid + i * _NW

        def issue(i):
            rows, lab, labq, sem_r, sem_l = bufs[i % 2]
            c = chunk_id(i)

            @pl.when(c < _N_CHUNKS)
            def _():
                pltpu.async_copy(z_hbm.at[pl.ds(c * _CHUNK, _CHUNK)], rows, sem_r)
                pltpu.async_copy(y_hbm.at[pl.ds(c * _CHUNK, _CHUNK)], lab, sem_l)

        def consume(i):
            rows, lab, labq, sem_r, sem_l = bufs[i % 2]
            c = chunk_id(i)

            @pl.when(c < _N_CHUNKS)
            def _():
                pltpu.make_async_copy(y_hbm.at[pl.ds(c * _CHUNK, _CHUNK)], lab, sem_l).wait()
                for j in range(_N_GROUPS):
                    for t in range(_GROUP // _NS):
                        labq[j, pl.ds(t * _NS, _NS)] = (
                            lab[pl.ds(j * _GROUP + t * _NS, _NS)])
                pltpu.make_async_copy(z_hbm.at[pl.ds(c * _CHUNK, _CHUNK)], rows, sem_r).wait()
                descs = []
                for j in range(_N_GROUPS):
                    idx = labq.at[j]
                    dst = acc if j % 2 == 0 else acc2
                    descs.append(pltpu.async_copy(
                        rows.at[pl.ds(j * _GROUP, _GROUP)], dst.at[idx],
                        sem_sc, add=True))
                    descs.append(pltpu.async_copy(
                        ones_buf, cacc.at[idx], sem_sc, add=True))
                for dsc in descs:
                    dsc.wait()

        issue(0)
        for i in range(_CHUNKS_PER_TILE):
            if i + 1 < _CHUNKS_PER_TILE:
                issue(i + 1)
            consume(i)

        plsc.subcore_barrier()

        @pl.when(s_idx == 0)
        def _():
            pltpu.sync_copy(acc, sums_hbm.at[c_idx])
            pltpu.sync_copy(acc2, sums_hbm.at[_NC + c_idx])
            pltpu.sync_copy(cacc, cnts_hbm.at[c_idx])

    return k(z_support, y3d)


def _tc_body(psums_ref, pcnts_ref, q_ref, tau_ref, out_ref):
    sums = ((psums_ref[0] + psums_ref[1])
            + (psums_ref[2] + psums_ref[3]))
    cnts = pcnts_ref[0] + pcnts_ref[1]
    cnt = cnts[:, 0:1]
    protos = sums / cnt
    q = q_ref[...]
    qn = jnp.sum(q * q, axis=1, keepdims=True)
    pn = jnp.sum(protos * protos, axis=1)[None, :]
    cross = lax.dot_general(q, protos, (((1,), (1,)), ((), ())))
    out_ref[...] = (-tau_ref[0, 0]) * (qn - 2.0 * cross + pn)


def kernel(z_support, y_support, z_query, tau):
    psums, pcnts = _sc_segment_sums(z_support, y_support)
    tau2d = jnp.asarray(tau, jnp.float32).reshape(1, 1)
    logits = pl.pallas_call(
        _tc_body,
        out_shape=jax.ShapeDtypeStruct((z_query.shape[0], _N_CLASSES), jnp.float32),
        in_specs=[
            pl.BlockSpec(memory_space=pltpu.VMEM),
            pl.BlockSpec(memory_space=pltpu.VMEM),
            pl.BlockSpec(memory_space=pltpu.VMEM),
            pl.BlockSpec(memory_space=pltpu.SMEM),
        ],
        out_specs=pl.BlockSpec(memory_space=pltpu.VMEM),
    )(psums, pcnts, z_query, tau2d)
    return logits


# speedup vs baseline: 2.2088x; 1.0176x over previous
import functools

import jax
import jax.numpy as jnp
from jax import lax
from jax.experimental import pallas as pl
from jax.experimental.pallas import tpu as pltpu
from jax.experimental.pallas import tpu_sc as plsc

_NC = 2
_NS = 16
_NW = _NC * _NS

_N_ROWS = 100000
_D = 128
_N_CLASSES = 256
_CHUNK = 400
_GROUP = 80
_N_GROUPS = _CHUNK // _GROUP
_N_CHUNKS = _N_ROWS // _CHUNK
_CHUNKS_PER_TILE = -(-_N_CHUNKS // _NW)


def _sc_segment_sums(z_support, y3d):
    mesh = plsc.VectorSubcoreMesh(core_axis_name="c", subcore_axis_name="s")

    @functools.partial(
        pl.kernel,
        out_type=(
            jax.ShapeDtypeStruct((_NC, _N_CLASSES, _D), jnp.float32),
            jax.ShapeDtypeStruct((_NC, _N_CLASSES, _NS), jnp.float32),
        ),
        mesh=mesh,
        scratch_types=dict(
            rows0=pltpu.VMEM((_CHUNK, _D), jnp.float32),
            rows1=pltpu.VMEM((_CHUNK, _D), jnp.float32),
            lab0=pltpu.VMEM((_CHUNK,), jnp.int32),
            lab1=pltpu.VMEM((_CHUNK,), jnp.int32),
            labq0=pltpu.VMEM((_N_GROUPS, _GROUP), jnp.int32),
            labq1=pltpu.VMEM((_N_GROUPS, _GROUP), jnp.int32),
            ones_buf=pltpu.VMEM((_GROUP, _NS), jnp.float32),
            zrow=pltpu.VMEM((_NS, _D), jnp.float32),
            zrow16=pltpu.VMEM((_NS, _NS), jnp.float32),
            acc=pltpu.VMEM_SHARED((_N_CLASSES, _D), jnp.float32),
            cacc=pltpu.VMEM_SHARED((_N_CLASSES, _NS), jnp.float32),
            sem_r0=pltpu.SemaphoreType.DMA,
            sem_r1=pltpu.SemaphoreType.DMA,
            sem_l0=pltpu.SemaphoreType.DMA,
            sem_l1=pltpu.SemaphoreType.DMA,
            sem_sc0=pltpu.SemaphoreType.DMA,
            sem_sc1=pltpu.SemaphoreType.DMA,
        ),
    )
    def k(z_hbm, y_hbm, sums_hbm, cnts_hbm, *, rows0, rows1, lab0, lab1,
          labq0, labq1, ones_buf, zrow, zrow16, acc, cacc, sem_r0, sem_r1,
          sem_l0, sem_l1, sem_sc0, sem_sc1):
        c_idx = lax.axis_index("c")
        s_idx = lax.axis_index("s")
        wid = s_idx * _NC + c_idx

        zero16 = jnp.zeros((_NS,), jnp.float32)
        one16 = jnp.ones((_NS,), jnp.float32)

        def fill_zrow(r, carry):
            for kk in range(_D // _NS):
                zrow[r, pl.ds(kk * _NS, _NS)] = zero16
            zrow16[r, :] = zero16
            return carry

        lax.fori_loop(0, _NS, fill_zrow, 0)

        def fill_ones(r, carry):
            ones_buf[r, :] = one16
            return carry

        lax.fori_loop(0, _GROUP, fill_ones, 0)

        pltpu.sync_copy(zrow, acc.at[pl.ds(s_idx * _NS, _NS)])
        pltpu.sync_copy(zrow16, cacc.at[pl.ds(s_idx * _NS, _NS)])
        plsc.subcore_barrier()

        bufs = ((rows0, lab0, labq0, sem_r0, sem_l0, sem_sc0),
                (rows1, lab1, labq1, sem_r1, sem_l1, sem_sc1))

        def chunk_id(i):
            return wid + i * _NW

        def scatter_descs(i):
            rows, lab, labq, sem_r, sem_l, sem_sc = bufs[i % 2]
            descs = []
            for j in range(_N_GROUPS):
                idx = labq.at[j]
                descs.append(pltpu.make_async_copy(
                    rows.at[pl.ds(j * _GROUP, _GROUP)], acc.at[idx], sem_sc))
                descs.append(pltpu.make_async_copy(ones_buf, cacc.at[idx], sem_sc))
            return descs

        def drain(i):
            c = chunk_id(i)

            @pl.when(c < _N_CHUNKS)
            def _():
                for dsc in scatter_descs(i):
                    dsc.wait()

        def issue(i):
            rows, lab, labq, sem_r, sem_l, sem_sc = bufs[i % 2]
            c = chunk_id(i)

            @pl.when(c < _N_CHUNKS)
            def _():
                pltpu.async_copy(z_hbm.at[pl.ds(c * _CHUNK, _CHUNK)], rows, sem_r)
                pltpu.async_copy(y_hbm.at[pl.ds(c * _CHUNK, _CHUNK)], lab, sem_l)

        def consume(i):
            rows, lab, labq, sem_r, sem_l, sem_sc = bufs[i % 2]
            c = chunk_id(i)

            @pl.when(c < _N_CHUNKS)
            def _():
                pltpu.make_async_copy(y_hbm.at[pl.ds(c * _CHUNK, _CHUNK)], lab, sem_l).wait()
                for j in range(_N_GROUPS):
                    for t in range(_GROUP // _NS):
                        labq[j, pl.ds(t * _NS, _NS)] = (
                            lab[pl.ds(j * _GROUP + t * _NS, _NS)])
                pltpu.make_async_copy(z_hbm.at[pl.ds(c * _CHUNK, _CHUNK)], rows, sem_r).wait()
                for j in range(_N_GROUPS):
                    idx = labq.at[j]
                    pltpu.async_copy(
                        rows.at[pl.ds(j * _GROUP, _GROUP)], acc.at[idx],
                        sem_sc, add=True)
                    pltpu.async_copy(ones_buf, cacc.at[idx], sem_sc, add=True)

        issue(0)
        for i in range(_CHUNKS_PER_TILE):
            if i + 1 < _CHUNKS_PER_TILE:
                if i >= 1:
                    drain(i - 1)
                issue(i + 1)
            consume(i)
        drain(_CHUNKS_PER_TILE - 2)
        drain(_CHUNKS_PER_TILE - 1)

        plsc.subcore_barrier()

        @pl.when(s_idx == 0)
        def _():
            pltpu.sync_copy(acc, sums_hbm.at[c_idx])
            pltpu.sync_copy(cacc, cnts_hbm.at[c_idx])

    return k(z_support, y3d)


def _tc_body(psums_ref, pcnts_ref, q_ref, tau_ref, out_ref):
    sums = psums_ref[0] + psums_ref[1]
    cnts = pcnts_ref[0] + pcnts_ref[1]
    cnt = cnts[:, 0:1]
    protos = sums / cnt
    q = q_ref[...]
    qn = jnp.sum(q * q, axis=1, keepdims=True)
    pn = jnp.sum(protos * protos, axis=1)[None, :]
    cross = lax.dot_general(q, protos, (((1,), (1,)), ((), ())))
    out_ref[...] = (-tau_ref[0, 0]) * (qn - 2.0 * cross + pn)


def kernel(z_support, y_support, z_query, tau):
    psums, pcnts = _sc_segment_sums(z_support, y_support)
    tau2d = jnp.asarray(tau, jnp.float32).reshape(1, 1)
    logits = pl.pallas_call(
        _tc_body,
        out_shape=jax.ShapeDtypeStruct((z_query.shape[0], _N_CLASSES), jnp.float32),
        in_specs=[
            pl.BlockSpec(memory_space=pltpu.VMEM),
            pl.BlockSpec(memory_space=pltpu.VMEM),
            pl.BlockSpec(memory_space=pltpu.VMEM),
            pl.BlockSpec(memory_space=pltpu.SMEM),
        ],
        out_specs=pl.BlockSpec(memory_space=pltpu.VMEM),
    )(psums, pcnts, z_query, tau2d)
    return logits
